# baseline (speedup 1.0000x reference)
import functools

import jax
import jax.numpy as jnp
from jax import lax
from jax.experimental import pallas as pl
from jax.experimental.pallas import tpu as pltpu
from jax.experimental.pallas import tpu_sc as plsc

VOCAB = 1000000
HID = 64
MAXLEN = 512
B = 1024
S = 512
N = B * S

NC = 2
NS = 16
NW = NC * NS

NSPLIT = 4
NH = N // NSPLIT
CHUNK = 1024
KSUB = CHUNK // 128
PER_W = NH // NW
NITER = PER_W // CHUNK
IDX_ROWS = PER_W // 128

Bb = 32
NBLK = B // Bb // NSPLIT


def _sc_gather(ids2d, table, half):
    mesh = plsc.VectorSubcoreMesh(core_axis_name="c", subcore_axis_name="s")

    @functools.partial(
        pl.kernel,
        out_type=jax.ShapeDtypeStruct((NH, 128), jnp.float32),
        mesh=mesh,
        scratch_types=[
            pltpu.VMEM((IDX_ROWS, 128), jnp.int32),
            pltpu.VMEM((CHUNK, HID), jnp.float32),
            pltpu.SemaphoreType.DMA,
        ],
        compiler_params=pltpu.CompilerParams(use_tc_tiling_on_sc=False),
    )
    def k(ids_hbm, table_hbm, out_hbm, idx_v, rows_v, sem):
        wid = lax.axis_index("s") * NC + lax.axis_index("c")
        idx_base = pl.multiple_of(half * (NH // 128) + wid * IDX_ROWS, IDX_ROWS)
        pltpu.sync_copy(ids_hbm.at[pl.ds(idx_base, IDX_ROWS)], idx_v)

        def body(i, _):
            base = pl.multiple_of(wid * PER_W + i * CHUNK, CHUNK)
            handles = []
            for ksub in range(KSUB):
                handles.append(pltpu.async_copy(
                    table_hbm.at[idx_v.at[i * KSUB + ksub]],
                    rows_v.at[pl.ds(ksub * 128, 128)],
                    sem,
                ))
            for h in handles:
                h.wait()
            pltpu.sync_copy(rows_v, out_hbm.at[pl.ds(base, CHUNK), pl.ds(0, HID)])
            return ()

        lax.fori_loop(0, NITER, body, ())

    return k(ids2d, table)


def _tc_ln_body(g_ref, tt_ref, pos_ref, seg_ref, gamma_ref, beta_ref, o_ref):
    x = g_ref[...][:, :HID].reshape(Bb, S, HID)
    tt = tt_ref[...]
    pos = pos_ref[...]
    seg = seg_ref[...]
    x = x + pos[None, :, :]
    x = x + jnp.where((tt[:, :, None] == 1), seg[1][None, None, :],
                      seg[0][None, None, :])
    mean = jnp.mean(x, axis=-1, keepdims=True)
    xc = x - mean
    var = jnp.mean(xc * xc, axis=-1, keepdims=True)
    y = xc * lax.rsqrt(var + 1e-5)
    o_ref[...] = y * gamma_ref[...][None, None, :] + beta_ref[...][None, None, :]


def _tc_ln_half(g, tt, pos, seg, gamma, beta, half, prev=None):
    def ln_body(g_ref, tt_ref, pos_ref, seg_ref, gamma_ref, beta_ref, *rest):
        _tc_ln_body(g_ref, tt_ref, pos_ref, seg_ref, gamma_ref, beta_ref,
                    rest[-1])

    boff = half * NBLK
    in_specs = [
        pl.BlockSpec((Bb * S, 128), lambda i: (i, 0)),
        pl.BlockSpec((Bb, S), lambda i: (i + boff, 0)),
        pl.BlockSpec((S, HID), lambda i: (0, 0)),
        pl.BlockSpec((2, HID), lambda i: (0, 0)),
        pl.BlockSpec((HID,), lambda i: (0,)),
        pl.BlockSpec((HID,), lambda i: (0,)),
    ]
    args = [g, tt, pos, seg, gamma, beta]
    aliases = {}
    if prev is not None:
        in_specs.append(pl.BlockSpec((Bb, S, HID), lambda i: (0, 0, 0)))
        args.append(prev)
        aliases = {6: 0}
    return pl.pallas_call(
        ln_body,
        grid=(NBLK,),
        in_specs=in_specs,
        out_specs=pl.BlockSpec((Bb, S, HID), lambda i: (i + boff, 0, 0)),
        out_shape=jax.ShapeDtypeStruct((B, S, HID), jnp.float32),
        input_output_aliases=aliases,
    )(*args)


def kernel(input_ids, token_type_ids, token_table, pos_table, seg_table, gamma, beta):
    ids2d = input_ids.reshape(N // 128, 128)
    parts = [_sc_gather(ids2d, token_table, h) for h in range(NSPLIT)]
    out = None
    for h, gh in enumerate(parts):
        out = _tc_ln_half(gh, token_type_ids, pos_table, seg_table, gamma,
                          beta, h, prev=out)
    return out


# speedup vs baseline: 1.0045x; 1.0045x over previous
import functools

import jax
import jax.numpy as jnp
from jax import lax
from jax.experimental import pallas as pl
from jax.experimental.pallas import tpu as pltpu
from jax.experimental.pallas import tpu_sc as plsc

VOCAB = 1000000
HID = 64
MAXLEN = 512
B = 1024
S = 512
N = B * S

NC = 2
NS = 16
NW = NC * NS

NSPLIT = 2
NH = N // NSPLIT
CHUNK = 1024
KSUB = CHUNK // 128
PER_W = NH // NW
NITER = PER_W // CHUNK
IDX_ROWS = PER_W // 128

Bb = 32
NBLK = B // Bb // NSPLIT


def _sc_gather(ids2d, table, half):
    mesh = plsc.VectorSubcoreMesh(core_axis_name="c", subcore_axis_name="s")

    @functools.partial(
        pl.kernel,
        out_type=jax.ShapeDtypeStruct((NH, 128), jnp.float32),
        mesh=mesh,
        scratch_types=[
            pltpu.VMEM((IDX_ROWS, 128), jnp.int32),
            pltpu.VMEM((CHUNK, HID), jnp.float32),
            pltpu.SemaphoreType.DMA,
        ],
        compiler_params=pltpu.CompilerParams(use_tc_tiling_on_sc=False),
    )
    def k(ids_hbm, table_hbm, out_hbm, idx_v, rows_v, sem):
        wid = lax.axis_index("s") * NC + lax.axis_index("c")
        idx_base = pl.multiple_of(half * (NH // 128) + wid * IDX_ROWS, IDX_ROWS)
        pltpu.sync_copy(ids_hbm.at[pl.ds(idx_base, IDX_ROWS)], idx_v)

        def body(i, _):
            base = pl.multiple_of(wid * PER_W + i * CHUNK, CHUNK)
            handles = []
            for ksub in range(KSUB):
                handles.append(pltpu.async_copy(
                    table_hbm.at[idx_v.at[i * KSUB + ksub]],
                    rows_v.at[pl.ds(ksub * 128, 128)],
                    sem,
                ))
            for h in handles:
                h.wait()
            pltpu.sync_copy(rows_v, out_hbm.at[pl.ds(base, CHUNK), pl.ds(0, HID)])
            return ()

        lax.fori_loop(0, NITER, body, ())

    return k(ids2d, table)


def _tc_ln_body(g_ref, tt_ref, pos_ref, seg_ref, gamma_ref, beta_ref, o_ref):
    x = g_ref[...][:, :HID].reshape(Bb, S, HID)
    tt = tt_ref[...]
    pos = pos_ref[...]
    seg = seg_ref[...]
    x = x + pos[None, :, :]
    x = x + jnp.where((tt[:, :, None] == 1), seg[1][None, None, :],
                      seg[0][None, None, :])
    mean = jnp.mean(x, axis=-1, keepdims=True)
    xc = x - mean
    var = jnp.mean(xc * xc, axis=-1, keepdims=True)
    y = xc * lax.rsqrt(var + 1e-5)
    o_ref[...] = y * gamma_ref[...][None, None, :] + beta_ref[...][None, None, :]


def _tc_ln_half(g, tt, pos, seg, gamma, beta, half, prev=None):
    def ln_body(g_ref, tt_ref, pos_ref, seg_ref, gamma_ref, beta_ref, *rest):
        _tc_ln_body(g_ref, tt_ref, pos_ref, seg_ref, gamma_ref, beta_ref,
                    rest[-1])

    boff = half * NBLK
    in_specs = [
        pl.BlockSpec((Bb * S, 128), lambda i: (i, 0)),
        pl.BlockSpec((Bb, S), lambda i: (i + boff, 0)),
        pl.BlockSpec((S, HID), lambda i: (0, 0)),
        pl.BlockSpec((2, HID), lambda i: (0, 0)),
        pl.BlockSpec((HID,), lambda i: (0,)),
        pl.BlockSpec((HID,), lambda i: (0,)),
    ]
    args = [g, tt, pos, seg, gamma, beta]
    aliases = {}
    if prev is not None:
        in_specs.append(pl.BlockSpec((Bb, S, HID), lambda i: (0, 0, 0)))
        args.append(prev)
        aliases = {6: 0}
    return pl.pallas_call(
        ln_body,
        grid=(NBLK,),
        in_specs=in_specs,
        out_specs=pl.BlockSpec((Bb, S, HID), lambda i: (i + boff, 0, 0)),
        out_shape=jax.ShapeDtypeStruct((B, S, HID), jnp.float32),
        input_output_aliases=aliases,
    )(*args)


def kernel(input_ids, token_type_ids, token_table, pos_table, seg_table, gamma, beta):
    ids2d = input_ids.reshape(N // 128, 128)
    parts = [_sc_gather(ids2d, token_table, h) for h in range(NSPLIT)]
    out = None
    for h, gh in enumerate(parts):
        out = _tc_ln_half(gh, token_type_ids, pos_table, seg_table, gamma,
                          beta, h, prev=out)
    return out
